# Initial kernel scaffold; baseline (speedup 1.0000x reference)
import jax
import jax.numpy as jnp
from jax.experimental import pallas as pl


def kernel(U, Wq, bq, Wk, bk):
    raise NotImplementedError("write your pallas kernel here")


# speedup vs baseline: 4.7807x; 4.7807x over previous
import functools
import math

import jax
import jax.numpy as jnp
from jax import lax
from jax.experimental import pallas as pl

N, T, D = 4096, 16, 128
KEY_DIM = 64
TOPK = 8
BLK = 256
NBLK = N // BLK
SCALE = 1.0 / math.sqrt(KEY_DIM)


def _pool_proj_body(u_ref, wqt_ref, bq_ref, wkt_ref, bk_ref, q_ref, k_ref):
    pool = jnp.mean(u_ref[...], axis=1)
    q_ref[...] = jnp.dot(pool, wqt_ref[...],
                         preferred_element_type=jnp.float32) + bq_ref[...]
    k_ref[...] = jnp.dot(pool, wkt_ref[...],
                         preferred_element_type=jnp.float32) + bk_ref[...]


def _topk_body(q_ref, k_ref, cols_ref, vals_ref):
    s = lax.dot_general(q_ref[...], k_ref[...],
                        (((1,), (1,)), ((), ())),
                        preferred_element_type=jnp.float32) * SCALE
    col_iota = lax.broadcasted_iota(jnp.int32, (BLK, N), 1)

    vals = []
    idxs = []
    for _ in range(TOPK):
        m = jnp.max(s, axis=1, keepdims=True)
        idx = jnp.min(jnp.where(s == m, col_iota, N), axis=1,
                      keepdims=True)
        vals.append(m)
        idxs.append(idx)
        s = jnp.where(col_iota == idx, -jnp.inf, s)
    v = jnp.concatenate(vals, axis=1)
    ix = jnp.concatenate(idxs, axis=1)

    e = jnp.exp(v - v[:, 0:1])
    p = e / jnp.sum(e, axis=1, keepdims=True)

    rank = jnp.zeros((BLK, TOPK), jnp.int32)
    for l in range(TOPK):
        rank = rank + (ix[:, l:l + 1] < ix).astype(jnp.int32)
    out_ix = []
    out_v = []
    for pos in range(TOPK):
        sel = rank == pos
        out_ix.append(jnp.sum(jnp.where(sel, ix, 0), axis=1, keepdims=True))
        out_v.append(jnp.sum(jnp.where(sel, p, 0.0), axis=1, keepdims=True))
    cols_ref[...] = jnp.concatenate(out_ix, axis=1)
    vals_ref[...] = jnp.concatenate(out_v, axis=1)


@jax.jit
def kernel(U, Wq, bq, Wk, bk):
    q, k = pl.pallas_call(
        _pool_proj_body,
        grid=(NBLK,),
        in_specs=[
            pl.BlockSpec((BLK, T, D), lambda i: (i, 0, 0)),
            pl.BlockSpec((D, KEY_DIM), lambda i: (0, 0)),
            pl.BlockSpec((1, KEY_DIM), lambda i: (0, 0)),
            pl.BlockSpec((D, KEY_DIM), lambda i: (0, 0)),
            pl.BlockSpec((1, KEY_DIM), lambda i: (0, 0)),
        ],
        out_specs=[
            pl.BlockSpec((BLK, KEY_DIM), lambda i: (i, 0)),
            pl.BlockSpec((BLK, KEY_DIM), lambda i: (i, 0)),
        ],
        out_shape=[
            jax.ShapeDtypeStruct((N, KEY_DIM), jnp.float32),
            jax.ShapeDtypeStruct((N, KEY_DIM), jnp.float32),
        ],
    )(U, Wq.T, bq.reshape(1, KEY_DIM), Wk.T, bk.reshape(1, KEY_DIM))

    cols, vals = pl.pallas_call(
        _topk_body,
        grid=(NBLK,),
        in_specs=[
            pl.BlockSpec((BLK, KEY_DIM), lambda i: (i, 0)),
            pl.BlockSpec((N, KEY_DIM), lambda i: (0, 0)),
        ],
        out_specs=[
            pl.BlockSpec((BLK, TOPK), lambda i: (i, 0)),
            pl.BlockSpec((BLK, TOPK), lambda i: (i, 0)),
        ],
        out_shape=[
            jax.ShapeDtypeStruct((N, TOPK), jnp.int32),
            jax.ShapeDtypeStruct((N, TOPK), jnp.float32),
        ],
    )(q, k)

    rows = jnp.repeat(jnp.arange(N, dtype=jnp.int32), TOPK)
    indices = jnp.stack([rows.astype(jnp.int64),
                         cols.reshape(-1).astype(jnp.int64)], axis=0)
    return indices, vals.reshape(-1)
